# Initial kernel scaffold; baseline (speedup 1.0000x reference)
import jax
import jax.numpy as jnp
from jax.experimental import pallas as pl


def kernel(inputs_embeds, token_type_ids, position_ids, pos_table, type_table, ln_weight, ln_bias):
    raise NotImplementedError("write your pallas kernel here")


# speedup vs baseline: 1.9287x; 1.9287x over previous
import functools

import jax
import jax.numpy as jnp
from jax import lax
from jax.experimental import pallas as pl
from jax.experimental.pallas import tpu as pltpu
from jax.experimental.pallas import tpu_sc as plsc

EPS = 1e-12

_NUM_CORES = 2
_NUM_SUBCORES = 16
_NUM_WORKERS = _NUM_CORES * _NUM_SUBCORES


def _sc_gather(table, idx_flat):
    n, = idx_flat.shape
    h = table.shape[1]
    rows_per_worker = n // _NUM_WORKERS
    chunk = 32
    n_chunks = rows_per_worker // chunk
    mesh = plsc.VectorSubcoreMesh(
        core_axis_name="c", subcore_axis_name="s",
        num_cores=_NUM_CORES, num_subcores=_NUM_SUBCORES)

    @functools.partial(
        pl.kernel,
        mesh=mesh,
        out_type=jax.ShapeDtypeStruct((n, h), jnp.float32),
        scratch_types=[
            pltpu.VMEM((chunk,), jnp.int32),
            pltpu.VMEM((chunk, h), jnp.float32),
            pltpu.SemaphoreType.DMA,
        ],
    )
    def gather_kernel(table_hbm, idx_hbm, out_hbm, idx_v, rows_v, sem):
        wid = lax.axis_index("s") * _NUM_CORES + lax.axis_index("c")
        base = wid * rows_per_worker

        def body(i, carry):
            off = base + i * chunk
            pltpu.sync_copy(idx_hbm.at[pl.ds(off, chunk)], idx_v)
            pltpu.async_copy(table_hbm.at[idx_v], rows_v, sem).wait()
            pltpu.sync_copy(rows_v, out_hbm.at[pl.ds(off, chunk)])
            return carry

        lax.fori_loop(0, n_chunks, body, 0)

    return gather_kernel(table, idx_flat)


def _tc_fused_ln(embeds2, pos2, tids3, type_table, w2, b2):
    n, h = embeds2.shape
    block_rows = 256
    n_blocks = n // block_rows

    def body(emb_ref, pos_ref, tid_ref, tt_ref, w_ref, b_ref, out_ref):
        e = emb_ref[...] + pos_ref[...]
        t = tid_ref[0, 0, :].reshape(block_rows, 1)
        te = jnp.where(t == 1, tt_ref[1, :][None, :], tt_ref[0, :][None, :])
        e = e + te
        u = jnp.mean(e, axis=-1, keepdims=True)
        d = e - u
        s = jnp.mean(d * d, axis=-1, keepdims=True)
        x = d * lax.rsqrt(s + EPS)
        out_ref[...] = w_ref[...] * x + b_ref[...]

    return pl.pallas_call(
        body,
        grid=(n_blocks,),
        in_specs=[
            pl.BlockSpec((block_rows, h), lambda i: (i, 0)),
            pl.BlockSpec((block_rows, h), lambda i: (i, 0)),
            pl.BlockSpec((1, 1, block_rows), lambda i: (i, 0, 0)),
            pl.BlockSpec((2, h), lambda i: (0, 0)),
            pl.BlockSpec((1, h), lambda i: (0, 0)),
            pl.BlockSpec((1, h), lambda i: (0, 0)),
        ],
        out_specs=pl.BlockSpec((block_rows, h), lambda i: (i, 0)),
        out_shape=jax.ShapeDtypeStruct((n, h), jnp.float32),
    )(embeds2, pos2, tids3, type_table, w2, b2)


def kernel(inputs_embeds, token_type_ids, position_ids, pos_table,
           type_table, ln_weight, ln_bias):
    b, s, h = inputs_embeds.shape
    n = b * s
    embeds2 = inputs_embeds.reshape(n, h)
    pos_flat = position_ids.reshape(n).astype(jnp.int32)
    block_rows = 256
    tids3 = token_type_ids.reshape(n // block_rows, 1, block_rows).astype(jnp.int32)

    pos2 = _sc_gather(pos_table, pos_flat)
    out2 = _tc_fused_ln(embeds2, pos2, tids3, type_table,
                        ln_weight.reshape(1, h), ln_bias.reshape(1, h))
    return out2.reshape(b, s, h)


# speedup vs baseline: 2.1682x; 1.1242x over previous
import functools

import jax
import jax.numpy as jnp
from jax import lax
from jax.experimental import pallas as pl
from jax.experimental.pallas import tpu as pltpu
from jax.experimental.pallas import tpu_sc as plsc

EPS = 1e-12

_NUM_CORES = 2
_NUM_SUBCORES = 16
_NUM_WORKERS = _NUM_CORES * _NUM_SUBCORES


def _sc_gather(table, idx_flat):
    n, = idx_flat.shape
    h = table.shape[1]
    rows_per_worker = n // _NUM_WORKERS
    chunk = 32
    n_chunks = rows_per_worker // chunk
    mesh = plsc.VectorSubcoreMesh(
        core_axis_name="c", subcore_axis_name="s",
        num_cores=_NUM_CORES, num_subcores=_NUM_SUBCORES)

    @functools.partial(
        pl.kernel,
        mesh=mesh,
        out_type=jax.ShapeDtypeStruct((n, h), jnp.float32),
        scratch_types=[
            pltpu.VMEM((rows_per_worker,), jnp.int32),
            pltpu.VMEM((chunk, h), jnp.float32),
            pltpu.VMEM((chunk, h), jnp.float32),
            pltpu.SemaphoreType.DMA,
            pltpu.SemaphoreType.DMA,
        ],
    )
    def gather_kernel(table_hbm, idx_hbm, out_hbm, idx_v, rows_a, rows_b,
                      sem_a, sem_b):
        wid = lax.axis_index("s") * _NUM_CORES + lax.axis_index("c")
        base = wid * rows_per_worker
        pltpu.sync_copy(idx_hbm.at[pl.ds(base, rows_per_worker)], idx_v)
        pltpu.async_copy(table_hbm.at[idx_v.at[pl.ds(0, chunk)]], rows_a,
                         sem_a)

        def body(i, carry):
            even = lax.rem(i, 2) == 0
            more = i + 1 < n_chunks

            @pl.when(jnp.logical_and(even, more))
            def _():
                pltpu.async_copy(
                    table_hbm.at[idx_v.at[pl.ds((i + 1) * chunk, chunk)]],
                    rows_b, sem_b)

            @pl.when(jnp.logical_and(jnp.logical_not(even), more))
            def _():
                pltpu.async_copy(
                    table_hbm.at[idx_v.at[pl.ds((i + 1) * chunk, chunk)]],
                    rows_a, sem_a)

            @pl.when(even)
            def _():
                pltpu.make_async_copy(table_hbm.at[pl.ds(0, chunk)], rows_a,
                                      sem_a).wait()
                pltpu.sync_copy(rows_a, out_hbm.at[pl.ds(base + i * chunk,
                                                         chunk)])

            @pl.when(jnp.logical_not(even))
            def _():
                pltpu.make_async_copy(table_hbm.at[pl.ds(0, chunk)], rows_b,
                                      sem_b).wait()
                pltpu.sync_copy(rows_b, out_hbm.at[pl.ds(base + i * chunk,
                                                         chunk)])

            return carry

        lax.fori_loop(0, n_chunks, body, 0)

    return gather_kernel(table, idx_flat)


def _tc_fused_ln(embeds2, pos2, tids3, type_table, w2, b2):
    n, h = embeds2.shape
    block_rows = 256
    n_blocks = n // block_rows

    def body(emb_ref, pos_ref, tid_ref, tt_ref, w_ref, b_ref, out_ref):
        e = emb_ref[...] + pos_ref[...]
        t = tid_ref[0, 0, :].reshape(block_rows, 1)
        te = jnp.where(t == 1, tt_ref[1, :][None, :], tt_ref[0, :][None, :])
        e = e + te
        u = jnp.mean(e, axis=-1, keepdims=True)
        d = e - u
        s = jnp.mean(d * d, axis=-1, keepdims=True)
        x = d * lax.rsqrt(s + EPS)
        out_ref[...] = w_ref[...] * x + b_ref[...]

    return pl.pallas_call(
        body,
        grid=(n_blocks,),
        in_specs=[
            pl.BlockSpec((block_rows, h), lambda i: (i, 0)),
            pl.BlockSpec((block_rows, h), lambda i: (i, 0)),
            pl.BlockSpec((1, 1, block_rows), lambda i: (i, 0, 0)),
            pl.BlockSpec((2, h), lambda i: (0, 0)),
            pl.BlockSpec((1, h), lambda i: (0, 0)),
            pl.BlockSpec((1, h), lambda i: (0, 0)),
        ],
        out_specs=pl.BlockSpec((block_rows, h), lambda i: (i, 0)),
        out_shape=jax.ShapeDtypeStruct((n, h), jnp.float32),
    )(embeds2, pos2, tids3, type_table, w2, b2)


def kernel(inputs_embeds, token_type_ids, position_ids, pos_table,
           type_table, ln_weight, ln_bias):
    b, s, h = inputs_embeds.shape
    n = b * s
    embeds2 = inputs_embeds.reshape(n, h)
    pos_flat = position_ids.reshape(n).astype(jnp.int32)
    block_rows = 256
    tids3 = token_type_ids.reshape(n // block_rows, 1, block_rows).astype(jnp.int32)

    pos2 = _sc_gather(pos_table, pos_flat)
    out2 = _tc_fused_ln(embeds2, pos2, tids3, type_table,
                        ln_weight.reshape(1, h), ln_bias.reshape(1, h))
    return out2.reshape(b, s, h)


# speedup vs baseline: 2.4745x; 1.1413x over previous
import functools

import jax
import jax.numpy as jnp
from jax import lax
from jax.experimental import pallas as pl
from jax.experimental.pallas import tpu as pltpu
from jax.experimental.pallas import tpu_sc as plsc

EPS = 1e-12

_NUM_CORES = 2
_NUM_SUBCORES = 16
_NUM_WORKERS = _NUM_CORES * _NUM_SUBCORES


def _sc_gather(table, idx_flat):
    n, = idx_flat.shape
    h = table.shape[1]
    rows_per_worker = n // _NUM_WORKERS
    chunk = 32
    n_chunks = rows_per_worker // chunk
    mesh = plsc.VectorSubcoreMesh(
        core_axis_name="c", subcore_axis_name="s",
        num_cores=_NUM_CORES, num_subcores=_NUM_SUBCORES)

    @functools.partial(
        pl.kernel,
        mesh=mesh,
        out_type=jax.ShapeDtypeStruct((n, h), jnp.float32),
        scratch_types=[
            pltpu.VMEM((rows_per_worker,), jnp.int32),
            pltpu.VMEM((chunk, h), jnp.float32),
            pltpu.VMEM((chunk, h), jnp.float32),
            pltpu.SemaphoreType.DMA,
            pltpu.SemaphoreType.DMA,
        ],
    )
    def gather_kernel(table_hbm, idx_hbm, out_hbm, idx_v, rows_a, rows_b,
                      sem_a, sem_b):
        wid = lax.axis_index("s") * _NUM_CORES + lax.axis_index("c")
        base = wid * rows_per_worker
        pltpu.sync_copy(idx_hbm.at[pl.ds(base, rows_per_worker)], idx_v)
        pltpu.async_copy(table_hbm.at[idx_v.at[pl.ds(0, chunk)]], rows_a,
                         sem_a)

        def body(i, carry):
            even = lax.rem(i, 2) == 0
            more = i + 1 < n_chunks

            @pl.when(jnp.logical_and(even, more))
            def _():
                pltpu.async_copy(
                    table_hbm.at[idx_v.at[pl.ds((i + 1) * chunk, chunk)]],
                    rows_b, sem_b)

            @pl.when(jnp.logical_and(jnp.logical_not(even), more))
            def _():
                pltpu.async_copy(
                    table_hbm.at[idx_v.at[pl.ds((i + 1) * chunk, chunk)]],
                    rows_a, sem_a)

            @pl.when(even)
            def _():
                pltpu.make_async_copy(table_hbm.at[pl.ds(0, chunk)], rows_a,
                                      sem_a).wait()
                pltpu.sync_copy(rows_a, out_hbm.at[pl.ds(base + i * chunk,
                                                         chunk)])

            @pl.when(jnp.logical_not(even))
            def _():
                pltpu.make_async_copy(table_hbm.at[pl.ds(0, chunk)], rows_b,
                                      sem_b).wait()
                pltpu.sync_copy(rows_b, out_hbm.at[pl.ds(base + i * chunk,
                                                         chunk)])

            return carry

        lax.fori_loop(0, n_chunks, body, 0)

    return gather_kernel(table, idx_flat)


def _tc_fused_ln(embeds2, pos2, tids3, type_table, w2, b2):
    n, h = embeds2.shape
    block_rows = 512
    n_blocks = n // block_rows

    def body(emb_ref, pos_ref, tid_ref, tt_ref, w_ref, b_ref, out_ref):
        e = emb_ref[...] + pos_ref[...]
        t = tid_ref[0, 0, :].reshape(block_rows, 1)
        te = jnp.where(t == 1, tt_ref[1, :][None, :], tt_ref[0, :][None, :])
        e = e + te
        u = jnp.mean(e, axis=-1, keepdims=True)
        d = e - u
        s = jnp.mean(d * d, axis=-1, keepdims=True)
        x = d * lax.rsqrt(s + EPS)
        out_ref[...] = w_ref[...] * x + b_ref[...]

    return pl.pallas_call(
        body,
        grid=(n_blocks,),
        in_specs=[
            pl.BlockSpec((block_rows, h), lambda i: (i, 0)),
            pl.BlockSpec((block_rows, h), lambda i: (i, 0)),
            pl.BlockSpec((1, 1, block_rows), lambda i: (i, 0, 0)),
            pl.BlockSpec((2, h), lambda i: (0, 0)),
            pl.BlockSpec((1, h), lambda i: (0, 0)),
            pl.BlockSpec((1, h), lambda i: (0, 0)),
        ],
        out_specs=pl.BlockSpec((block_rows, h), lambda i: (i, 0)),
        out_shape=jax.ShapeDtypeStruct((n, h), jnp.float32),
    )(embeds2, pos2, tids3, type_table, w2, b2)


def kernel(inputs_embeds, token_type_ids, position_ids, pos_table,
           type_table, ln_weight, ln_bias):
    b, s, h = inputs_embeds.shape
    n = b * s
    embeds2 = inputs_embeds.reshape(n, h)
    pos_flat = position_ids.reshape(n).astype(jnp.int32)
    block_rows = 512
    tids3 = token_type_ids.reshape(n // block_rows, 1, block_rows).astype(jnp.int32)
    w2 = ln_weight.reshape(1, h)
    b2 = ln_bias.reshape(1, h)

    pos2 = _sc_gather(pos_table, pos_flat)
    out2 = _tc_fused_ln(embeds2, pos2, tids3, type_table, w2, b2)
    return out2.reshape(b, s, h)


# speedup vs baseline: 2.5399x; 1.0264x over previous
import functools

import jax
import jax.numpy as jnp
from jax import lax
from jax.experimental import pallas as pl
from jax.experimental.pallas import tpu as pltpu
from jax.experimental.pallas import tpu_sc as plsc

EPS = 1e-12

_NUM_CORES = 2
_NUM_SUBCORES = 16
_NUM_WORKERS = _NUM_CORES * _NUM_SUBCORES


def _sc_gather(table, idx_flat):
    n, = idx_flat.shape
    h = table.shape[1]
    rows_per_worker = n // _NUM_WORKERS
    chunk = 32
    n_chunks = rows_per_worker // chunk
    mesh = plsc.VectorSubcoreMesh(
        core_axis_name="c", subcore_axis_name="s",
        num_cores=_NUM_CORES, num_subcores=_NUM_SUBCORES)

    @functools.partial(
        pl.kernel,
        mesh=mesh,
        out_type=jax.ShapeDtypeStruct((n, h), jnp.float32),
        scratch_types=[
            pltpu.VMEM((rows_per_worker,), jnp.int32),
            pltpu.VMEM((chunk, h), jnp.float32),
            pltpu.VMEM((chunk, h), jnp.float32),
            pltpu.SemaphoreType.DMA,
            pltpu.SemaphoreType.DMA,
        ],
    )
    def gather_kernel(table_hbm, idx_hbm, out_hbm, idx_v, rows_a, rows_b,
                      sem_a, sem_b):
        wid = lax.axis_index("s") * _NUM_CORES + lax.axis_index("c")
        base = wid * rows_per_worker
        pltpu.sync_copy(idx_hbm.at[pl.ds(base, rows_per_worker)], idx_v)
        pltpu.async_copy(table_hbm.at[idx_v.at[pl.ds(0, chunk)]], rows_a,
                         sem_a)

        def body(i, carry):
            even = lax.rem(i, 2) == 0
            more = i + 1 < n_chunks

            @pl.when(jnp.logical_and(even, more))
            def _():
                pltpu.async_copy(
                    table_hbm.at[idx_v.at[pl.ds((i + 1) * chunk, chunk)]],
                    rows_b, sem_b)

            @pl.when(jnp.logical_and(jnp.logical_not(even), more))
            def _():
                pltpu.async_copy(
                    table_hbm.at[idx_v.at[pl.ds((i + 1) * chunk, chunk)]],
                    rows_a, sem_a)

            @pl.when(even)
            def _():
                pltpu.make_async_copy(table_hbm.at[pl.ds(0, chunk)], rows_a,
                                      sem_a).wait()
                pltpu.sync_copy(rows_a, out_hbm.at[pl.ds(base + i * chunk,
                                                         chunk)])

            @pl.when(jnp.logical_not(even))
            def _():
                pltpu.make_async_copy(table_hbm.at[pl.ds(0, chunk)], rows_b,
                                      sem_b).wait()
                pltpu.sync_copy(rows_b, out_hbm.at[pl.ds(base + i * chunk,
                                                         chunk)])

            return carry

        lax.fori_loop(0, n_chunks, body, 0)

    return gather_kernel(table, idx_flat)


def _tc_fused_ln(embeds2, pos2, tids3, type_table, w2, b2):
    n, h = embeds2.shape
    block_rows = 1024
    n_blocks = n // block_rows

    def body(emb_ref, pos_ref, tid_ref, tt_ref, w_ref, b_ref, out_ref):
        e = emb_ref[...] + pos_ref[...]
        t = tid_ref[0, 0, :].reshape(block_rows, 1)
        te = jnp.where(t == 1, tt_ref[1, :][None, :], tt_ref[0, :][None, :])
        e = e + te
        u = jnp.mean(e, axis=-1, keepdims=True)
        d = e - u
        s = jnp.mean(d * d, axis=-1, keepdims=True)
        x = d * lax.rsqrt(s + EPS)
        out_ref[...] = w_ref[...] * x + b_ref[...]

    return pl.pallas_call(
        body,
        grid=(n_blocks,),
        in_specs=[
            pl.BlockSpec((block_rows, h), lambda i: (i, 0)),
            pl.BlockSpec((block_rows, h), lambda i: (i, 0)),
            pl.BlockSpec((1, 1, block_rows), lambda i: (i, 0, 0)),
            pl.BlockSpec((2, h), lambda i: (0, 0)),
            pl.BlockSpec((1, h), lambda i: (0, 0)),
            pl.BlockSpec((1, h), lambda i: (0, 0)),
        ],
        out_specs=pl.BlockSpec((block_rows, h), lambda i: (i, 0)),
        out_shape=jax.ShapeDtypeStruct((n, h), jnp.float32),
    )(embeds2, pos2, tids3, type_table, w2, b2)


def kernel(inputs_embeds, token_type_ids, position_ids, pos_table,
           type_table, ln_weight, ln_bias):
    b, s, h = inputs_embeds.shape
    n = b * s
    embeds2 = inputs_embeds.reshape(n, h)
    pos_flat = position_ids.reshape(n).astype(jnp.int32)
    block_rows = 1024
    tids3 = token_type_ids.reshape(n // block_rows, 1, block_rows).astype(jnp.int32)
    w2 = ln_weight.reshape(1, h)
    b2 = ln_bias.reshape(1, h)

    pos2 = _sc_gather(pos_table, pos_flat)
    out2 = _tc_fused_ln(embeds2, pos2, tids3, type_table, w2, b2)
    return out2.reshape(b, s, h)
